# baseline (speedup 1.0000x reference)
import functools

import jax
import jax.numpy as jnp
from jax import lax
from jax.experimental import pallas as pl
from jax.experimental.pallas import tpu as pltpu
from jax.experimental.pallas import tpu_sc as plsc

STEPS = 100000
EMBED_DIM = 128
BATCH = 16384

_INFO = plsc.get_sparse_core_info()
_NC = _INFO.num_cores
_NS = _INFO.num_subcores
_NW = _NC * _NS
_B_PER_W = BATCH // _NW


_CHUNK = 64
_NCHUNK = _B_PER_W // _CHUNK
_IDX_HALF = _B_PER_W // 2


@functools.partial(
    pl.kernel,
    mesh=plsc.VectorSubcoreMesh(core_axis_name="c", subcore_axis_name="s"),
    out_type=jax.ShapeDtypeStruct((BATCH, EMBED_DIM), jnp.float32),
    scratch_types=[
        pltpu.VMEM((_B_PER_W,), jnp.int32),
        pltpu.VMEM((_NCHUNK, _CHUNK, EMBED_DIM), jnp.float32),
    ]
    + [pltpu.SemaphoreType.DMA] * 2
    + [pltpu.SemaphoreType.DMA] * _NCHUNK
    + [pltpu.SemaphoreType.DMA],
)
def _gather_rows(table_hbm, idx_hbm, out_hbm, idx_v, rows_v, *sems):
    isems, gsems, osem = sems[:2], sems[2 : 2 + _NCHUNK], sems[2 + _NCHUNK]
    wid = lax.axis_index("s") * _NC + lax.axis_index("c")
    base = wid * _B_PER_W
    idx_loads = [
        pltpu.async_copy(
            idx_hbm.at[pl.ds(base + h * _IDX_HALF, _IDX_HALF)],
            idx_v.at[pl.ds(h * _IDX_HALF, _IDX_HALF)],
            isems[h],
        )
        for h in range(2)
    ]
    gathers = []
    for h in range(2):
        idx_loads[h].wait()
        for c in range(h * _NCHUNK // 2, (h + 1) * _NCHUNK // 2):
            gathers.append(
                pltpu.async_copy(
                    table_hbm.at[idx_v.at[pl.ds(c * _CHUNK, _CHUNK)]],
                    rows_v.at[c],
                    gsems[c],
                )
            )
    stores = []
    for c in range(_NCHUNK):
        gathers[c].wait()
        stores.append(
            pltpu.async_copy(
                rows_v.at[c],
                out_hbm.at[pl.ds(base + c * _CHUNK, _CHUNK)],
                osem,
            )
        )
    for st in stores:
        st.wait()


def kernel(x, t, embeddings):
    out = _gather_rows(embeddings, t)
    return out[:, :, None, None]


# speedup vs baseline: 1.0213x; 1.0213x over previous
import functools

import jax
import jax.numpy as jnp
from jax import lax
from jax.experimental import pallas as pl
from jax.experimental.pallas import tpu as pltpu
from jax.experimental.pallas import tpu_sc as plsc

STEPS = 100000
EMBED_DIM = 128
BATCH = 16384

_INFO = plsc.get_sparse_core_info()
_NC = _INFO.num_cores
_NS = _INFO.num_subcores
_NW = _NC * _NS
_B_PER_W = BATCH // _NW


_CHUNK = 128
_NCHUNK = _B_PER_W // _CHUNK
_IDX_HALF = _B_PER_W // 2


@functools.partial(
    pl.kernel,
    mesh=plsc.VectorSubcoreMesh(core_axis_name="c", subcore_axis_name="s"),
    out_type=jax.ShapeDtypeStruct((BATCH, EMBED_DIM), jnp.float32),
    scratch_types=[
        pltpu.VMEM((_B_PER_W,), jnp.int32),
        pltpu.VMEM((_NCHUNK, _CHUNK, EMBED_DIM), jnp.float32),
    ]
    + [pltpu.SemaphoreType.DMA] * 2
    + [pltpu.SemaphoreType.DMA] * _NCHUNK
    + [pltpu.SemaphoreType.DMA],
)
def _gather_rows(table_hbm, idx_hbm, out_hbm, idx_v, rows_v, *sems):
    isems, gsems, osem = sems[:2], sems[2 : 2 + _NCHUNK], sems[2 + _NCHUNK]
    wid = lax.axis_index("s") * _NC + lax.axis_index("c")
    base = wid * _B_PER_W
    idx_loads = [
        pltpu.async_copy(
            idx_hbm.at[pl.ds(base + h * _IDX_HALF, _IDX_HALF)],
            idx_v.at[pl.ds(h * _IDX_HALF, _IDX_HALF)],
            isems[h],
        )
        for h in range(2)
    ]
    gathers = []
    for h in range(2):
        idx_loads[h].wait()
        for c in range(h * _NCHUNK // 2, (h + 1) * _NCHUNK // 2):
            gathers.append(
                pltpu.async_copy(
                    table_hbm.at[idx_v.at[pl.ds(c * _CHUNK, _CHUNK)]],
                    rows_v.at[c],
                    gsems[c],
                )
            )
    stores = []
    for c in range(_NCHUNK):
        gathers[c].wait()
        stores.append(
            pltpu.async_copy(
                rows_v.at[c],
                out_hbm.at[pl.ds(base + c * _CHUNK, _CHUNK)],
                osem,
            )
        )
    for st in stores:
        st.wait()


def kernel(x, t, embeddings):
    out = _gather_rows(embeddings, t)
    return out[:, :, None, None]
